# baseline (speedup 1.0000x reference)
import functools

import jax
import jax.numpy as jnp
from jax import lax
from jax.experimental import pallas as pl
from jax.experimental.pallas import tpu as pltpu
from jax.experimental.pallas import tpu_sc as plsc

B = 8
N = 2048
L = 16
NC = 2
NS = 16
NW = NC * NS
BPC = B // NC
WPB = NS // BPC
SLICE = N // WPB
CH = 4
GRP = CH * L
BIG = 3.0e38
EPS = 1e-12


def _sqrt16(x):
    i = plsc.bitcast(x, jnp.int32)
    i = jnp.int32(0x5F3759DF) - (i >> 1)
    y = plsc.bitcast(i, jnp.float32)
    xh = x * jnp.float32(0.5)
    for _ in range(3):
        y = y * (jnp.float32(1.5) - xh * y * y)
    return x * y


def _cd_body(px_h, py_h, pz_h, pxb_h, pyb_h, pzb_h,
             qx_h, qy_h, qz_h, qxb_h, qyb_h, qzb_h,
             out1_h, out2_h,
             pxb_v, pyb_v, pzb_v, p2_v,
             qxb_v, qyb_v, qzb_v, q2_v,
             tx_v, ty_v, tz_v, comb_v, s_v, sp_d2):
    c = lax.axis_index("c")
    s = lax.axis_index("s")
    wid = c * NS + s
    b = c * BPC + s // WPB
    qtr = s % WPB
    pbase = b * N + qtr * SLICE
    qbase = b * N

    pltpu.sync_copy(pxb_h.at[pl.ds(pbase, SLICE)], pxb_v)
    pltpu.sync_copy(pyb_h.at[pl.ds(pbase, SLICE)], pyb_v)
    pltpu.sync_copy(pzb_h.at[pl.ds(pbase, SLICE)], pzb_v)
    pltpu.sync_copy(qxb_h.at[pl.ds(qbase, N)], qxb_v)
    pltpu.sync_copy(qyb_h.at[pl.ds(qbase, N)], qyb_v)
    pltpu.sync_copy(qzb_h.at[pl.ds(qbase, N)], qzb_v)

    pltpu.sync_copy(qx_h.at[pl.ds(qbase, N)], tx_v)
    pltpu.sync_copy(qy_h.at[pl.ds(qbase, N)], ty_v)
    pltpu.sync_copy(qz_h.at[pl.ds(qbase, N)], tz_v)
    for ch in range(N // L):
        x = tx_v[pl.ds(ch * L, L)]
        y = ty_v[pl.ds(ch * L, L)]
        z = tz_v[pl.ds(ch * L, L)]
        q2_v[pl.ds(ch * L, L)] = x * x + y * y + z * z
    pltpu.sync_copy(px_h.at[pl.ds(pbase, SLICE)], tx_v.at[pl.ds(0, SLICE)])
    pltpu.sync_copy(py_h.at[pl.ds(pbase, SLICE)], ty_v.at[pl.ds(0, SLICE)])
    pltpu.sync_copy(pz_h.at[pl.ds(pbase, SLICE)], tz_v.at[pl.ds(0, SLICE)])
    for ch in range(SLICE // L):
        x = tx_v[pl.ds(ch * L, L)]
        y = ty_v[pl.ds(ch * L, L)]
        z = tz_v[pl.ds(ch * L, L)]
        p2_v[pl.ds(ch * L, L)] = x * x + y * y + z * z

    s1 = jnp.zeros((L,), jnp.float32)
    for g in range(SLICE // GRP):
        base = g * GRP
        m2 = jnp.float32(-2.0)
        pxs = [pxb_v[pl.ds(base + k * L, L)] * m2 for k in range(CH)]
        pys = [pyb_v[pl.ds(base + k * L, L)] * m2 for k in range(CH)]
        pzs = [pzb_v[pl.ds(base + k * L, L)] * m2 for k in range(CH)]
        p2s = [p2_v[pl.ds(base + k * L, L)] for k in range(CH)]
        init = tuple(jnp.full((L,), BIG, jnp.float32) for _ in range(CH))

        @plsc.parallel_loop(0, N, unroll=8, carry=init)
        def acc1(j, acc):
            idx = jnp.full((L,), j, jnp.int32)
            qx = plsc.load_gather(qxb_v, [idx])
            qy = plsc.load_gather(qyb_v, [idx])
            qz = plsc.load_gather(qzb_v, [idx])
            q2 = plsc.load_gather(q2_v, [idx])
            out = []
            for k in range(CH):
                d = p2s[k] + (q2 + pxs[k] * qx + pys[k] * qy + pzs[k] * qz)
                out.append(jnp.minimum(acc[k], d))
            return tuple(out)

        for k in range(CH):
            s1 = s1 + _sqrt16(jnp.maximum(acc1[k], EPS))
    s_v[...] = s1
    pltpu.sync_copy(s_v, out1_h.at[wid])

    for g in range(N // GRP):
        base = g * GRP
        m2 = jnp.float32(-2.0)
        qxs = [qxb_v[pl.ds(base + k * L, L)] * m2 for k in range(CH)]
        qys = [qyb_v[pl.ds(base + k * L, L)] * m2 for k in range(CH)]
        qzs = [qzb_v[pl.ds(base + k * L, L)] * m2 for k in range(CH)]
        q2s = [q2_v[pl.ds(base + k * L, L)] for k in range(CH)]
        init = tuple(jnp.full((L,), BIG, jnp.float32) for _ in range(CH))

        @plsc.parallel_loop(0, SLICE, unroll=4, carry=init)
        def acc2(i, acc):
            idx = jnp.full((L,), i, jnp.int32)
            px = plsc.load_gather(pxb_v, [idx])
            py = plsc.load_gather(pyb_v, [idx])
            pz = plsc.load_gather(pzb_v, [idx])
            p2 = plsc.load_gather(p2_v, [idx])
            out = []
            for k in range(CH):
                d = q2s[k] + (p2 + qxs[k] * px + qys[k] * py + qzs[k] * pz)
                out.append(jnp.minimum(acc[k], d))
            return tuple(out)

        for k in range(CH):
            qxb_v[pl.ds(base + k * L, L)] = acc2[k]
    pltpu.sync_copy(qxb_v, sp_d2.at[s])
    plsc.subcore_barrier()

    s0 = (s // WPB) * WPB
    off = qtr * SLICE
    for t in range(WPB):
        pltpu.sync_copy(sp_d2.at[s0 + t, pl.ds(off, SLICE)], comb_v.at[t])
    s2 = jnp.zeros((L,), jnp.float32)
    for ch in range(SLICE // L):
        d = comb_v[0, pl.ds(ch * L, L)]
        for t in range(1, WPB):
            d = jnp.minimum(d, comb_v[t, pl.ds(ch * L, L)])
        s2 = s2 + _sqrt16(jnp.maximum(d, EPS))
    s_v[...] = s2
    pltpu.sync_copy(s_v, out2_h.at[wid])


_cd_call = pl.kernel(
    _cd_body,
    out_type=[
        jax.ShapeDtypeStruct((NW, L), jnp.float32),
        jax.ShapeDtypeStruct((NW, L), jnp.float32),
    ],
    mesh=plsc.VectorSubcoreMesh(
        core_axis_name="c", subcore_axis_name="s",
        num_cores=NC, num_subcores=NS),
    scratch_types=[
        pltpu.VMEM((SLICE,), jnp.float32),
        pltpu.VMEM((SLICE,), jnp.float32),
        pltpu.VMEM((SLICE,), jnp.float32),
        pltpu.VMEM((SLICE,), jnp.float32),
        pltpu.VMEM((N,), jnp.float32),
        pltpu.VMEM((N,), jnp.float32),
        pltpu.VMEM((N,), jnp.float32),
        pltpu.VMEM((N,), jnp.float32),
        pltpu.VMEM((N,), jnp.float32),
        pltpu.VMEM((N,), jnp.float32),
        pltpu.VMEM((N,), jnp.float32),
        pltpu.VMEM((WPB, SLICE), jnp.float32),
        pltpu.VMEM((L,), jnp.float32),
        pltpu.VMEM_SHARED((NS, N), jnp.float32),
    ],
    compiler_params=pltpu.CompilerParams(needs_layout_passes=False),
)


def kernel(y_pred, y_true):
    p = jnp.transpose(y_pred, (2, 0, 1)).reshape(3, B * N)
    q = jnp.transpose(y_true, (2, 0, 1)).reshape(3, B * N)
    def _rb(x):
        i = lax.bitcast_convert_type(x, jnp.int32)
        r = (i + jnp.int32(0x7FFF) + ((i >> 16) & jnp.int32(1))) & jnp.int32(-65536)
        return lax.bitcast_convert_type(r, jnp.float32)

    pb = _rb(p)
    qb = _rb(q)
    out1, out2 = _cd_call(p[0], p[1], p[2], pb[0], pb[1], pb[2],
                          q[0], q[1], q[2], qb[0], qb[1], qb[2])
    inv = 1.0 / (B * N)
    return (jnp.sum(out1) * inv + jnp.sum(out2) * inv) * 0.5


# speedup vs baseline: 1.1110x; 1.1110x over previous
import functools

import jax
import jax.numpy as jnp
from jax import lax
from jax.experimental import pallas as pl
from jax.experimental.pallas import tpu as pltpu
from jax.experimental.pallas import tpu_sc as plsc

B = 8
N = 2048
L = 16
NC = 2
NS = 16
NW = NC * NS
BPC = B // NC
WPB = NS // BPC
SLICE = N // WPB
CH = 4
GRP = CH * L
BIG = 3.0e38
EPS = 1e-12


def _sqrt16(x):
    i = plsc.bitcast(x, jnp.int32)
    i = jnp.int32(0x5F3759DF) - (i >> 1)
    y = plsc.bitcast(i, jnp.float32)
    xh = x * jnp.float32(0.5)
    for _ in range(3):
        y = y * (jnp.float32(1.5) - xh * y * y)
    return x * y


def _cd_body(px_h, py_h, pz_h, pxb_h, pyb_h, pzb_h,
             qx_h, qy_h, qz_h, qxb_h, qyb_h, qzb_h,
             out1_h, out2_h,
             pxb_v, pyb_v, pzb_v, p2_v,
             qxb_v, qyb_v, qzb_v, q2_v,
             tx_v, ty_v, tz_v, comb_v, s_v, sp_d2):
    c = lax.axis_index("c")
    s = lax.axis_index("s")
    wid = c * NS + s
    b = c * BPC + s // WPB
    qtr = s % WPB
    pbase = b * N + qtr * SLICE
    qbase = b * N

    pltpu.sync_copy(pxb_h.at[pl.ds(pbase, SLICE)], pxb_v)
    pltpu.sync_copy(pyb_h.at[pl.ds(pbase, SLICE)], pyb_v)
    pltpu.sync_copy(pzb_h.at[pl.ds(pbase, SLICE)], pzb_v)
    pltpu.sync_copy(qxb_h.at[pl.ds(qbase, N)], qxb_v)
    pltpu.sync_copy(qyb_h.at[pl.ds(qbase, N)], qyb_v)
    pltpu.sync_copy(qzb_h.at[pl.ds(qbase, N)], qzb_v)

    pltpu.sync_copy(qx_h.at[pl.ds(qbase, N)], tx_v)
    pltpu.sync_copy(qy_h.at[pl.ds(qbase, N)], ty_v)
    pltpu.sync_copy(qz_h.at[pl.ds(qbase, N)], tz_v)
    for ch in range(N // L):
        x = tx_v[pl.ds(ch * L, L)]
        y = ty_v[pl.ds(ch * L, L)]
        z = tz_v[pl.ds(ch * L, L)]
        q2_v[pl.ds(ch * L, L)] = x * x + y * y + z * z
    pltpu.sync_copy(px_h.at[pl.ds(pbase, SLICE)], tx_v.at[pl.ds(0, SLICE)])
    pltpu.sync_copy(py_h.at[pl.ds(pbase, SLICE)], ty_v.at[pl.ds(0, SLICE)])
    pltpu.sync_copy(pz_h.at[pl.ds(pbase, SLICE)], tz_v.at[pl.ds(0, SLICE)])
    for ch in range(SLICE // L):
        x = tx_v[pl.ds(ch * L, L)]
        y = ty_v[pl.ds(ch * L, L)]
        z = tz_v[pl.ds(ch * L, L)]
        p2_v[pl.ds(ch * L, L)] = x * x + y * y + z * z

    s1 = jnp.zeros((L,), jnp.float32)
    for g in range(SLICE // GRP):
        base = g * GRP
        m2 = jnp.float32(-2.0)
        pxs = [pxb_v[pl.ds(base + k * L, L)] * m2 for k in range(CH)]
        pys = [pyb_v[pl.ds(base + k * L, L)] * m2 for k in range(CH)]
        pzs = [pzb_v[pl.ds(base + k * L, L)] * m2 for k in range(CH)]
        p2s = [p2_v[pl.ds(base + k * L, L)] for k in range(CH)]
        init = tuple(jnp.full((L,), BIG, jnp.float32) for _ in range(CH))

        @plsc.parallel_loop(0, N, unroll=4, carry=init)
        def acc1(j, acc):
            idx = jnp.full((L,), j, jnp.int32)
            qx = plsc.load_gather(qxb_v, [idx])
            qy = plsc.load_gather(qyb_v, [idx])
            qz = plsc.load_gather(qzb_v, [idx])
            q2 = plsc.load_gather(q2_v, [idx])
            out = []
            for k in range(CH):
                d = p2s[k] + (q2 + pxs[k] * qx + pys[k] * qy + pzs[k] * qz)
                out.append(jnp.minimum(acc[k], d))
            return tuple(out)

        for k in range(CH):
            s1 = s1 + _sqrt16(jnp.maximum(acc1[k], EPS))
    s_v[...] = s1
    pltpu.sync_copy(s_v, out1_h.at[wid])

    for g in range(N // GRP):
        base = g * GRP
        m2 = jnp.float32(-2.0)
        qxs = [qxb_v[pl.ds(base + k * L, L)] * m2 for k in range(CH)]
        qys = [qyb_v[pl.ds(base + k * L, L)] * m2 for k in range(CH)]
        qzs = [qzb_v[pl.ds(base + k * L, L)] * m2 for k in range(CH)]
        q2s = [q2_v[pl.ds(base + k * L, L)] for k in range(CH)]
        init = tuple(jnp.full((L,), BIG, jnp.float32) for _ in range(CH))

        @plsc.parallel_loop(0, SLICE, unroll=4, carry=init)
        def acc2(i, acc):
            idx = jnp.full((L,), i, jnp.int32)
            px = plsc.load_gather(pxb_v, [idx])
            py = plsc.load_gather(pyb_v, [idx])
            pz = plsc.load_gather(pzb_v, [idx])
            p2 = plsc.load_gather(p2_v, [idx])
            out = []
            for k in range(CH):
                d = q2s[k] + (p2 + qxs[k] * px + qys[k] * py + qzs[k] * pz)
                out.append(jnp.minimum(acc[k], d))
            return tuple(out)

        for k in range(CH):
            qxb_v[pl.ds(base + k * L, L)] = acc2[k]
    pltpu.sync_copy(qxb_v, sp_d2.at[s])
    plsc.subcore_barrier()

    s0 = (s // WPB) * WPB
    off = qtr * SLICE
    for t in range(WPB):
        pltpu.sync_copy(sp_d2.at[s0 + t, pl.ds(off, SLICE)], comb_v.at[t])
    s2 = jnp.zeros((L,), jnp.float32)
    for ch in range(SLICE // L):
        d = comb_v[0, pl.ds(ch * L, L)]
        for t in range(1, WPB):
            d = jnp.minimum(d, comb_v[t, pl.ds(ch * L, L)])
        s2 = s2 + _sqrt16(jnp.maximum(d, EPS))
    s_v[...] = s2
    pltpu.sync_copy(s_v, out2_h.at[wid])


_cd_call = pl.kernel(
    _cd_body,
    out_type=[
        jax.ShapeDtypeStruct((NW, L), jnp.float32),
        jax.ShapeDtypeStruct((NW, L), jnp.float32),
    ],
    mesh=plsc.VectorSubcoreMesh(
        core_axis_name="c", subcore_axis_name="s",
        num_cores=NC, num_subcores=NS),
    scratch_types=[
        pltpu.VMEM((SLICE,), jnp.float32),
        pltpu.VMEM((SLICE,), jnp.float32),
        pltpu.VMEM((SLICE,), jnp.float32),
        pltpu.VMEM((SLICE,), jnp.float32),
        pltpu.VMEM((N,), jnp.float32),
        pltpu.VMEM((N,), jnp.float32),
        pltpu.VMEM((N,), jnp.float32),
        pltpu.VMEM((N,), jnp.float32),
        pltpu.VMEM((N,), jnp.float32),
        pltpu.VMEM((N,), jnp.float32),
        pltpu.VMEM((N,), jnp.float32),
        pltpu.VMEM((WPB, SLICE), jnp.float32),
        pltpu.VMEM((L,), jnp.float32),
        pltpu.VMEM_SHARED((NS, N), jnp.float32),
    ],
    compiler_params=pltpu.CompilerParams(needs_layout_passes=False),
)


def kernel(y_pred, y_true):
    p = jnp.transpose(y_pred, (2, 0, 1)).reshape(3, B * N)
    q = jnp.transpose(y_true, (2, 0, 1)).reshape(3, B * N)
    def _rb(x):
        i = lax.bitcast_convert_type(x, jnp.int32)
        r = (i + jnp.int32(0x7FFF) + ((i >> 16) & jnp.int32(1))) & jnp.int32(-65536)
        return lax.bitcast_convert_type(r, jnp.float32)

    pb = _rb(p)
    qb = _rb(q)
    out1, out2 = _cd_call(p[0], p[1], p[2], pb[0], pb[1], pb[2],
                          q[0], q[1], q[2], qb[0], qb[1], qb[2])
    inv = 1.0 / (B * N)
    return (jnp.sum(out1) * inv + jnp.sum(out2) * inv) * 0.5


# speedup vs baseline: 4.3585x; 3.9230x over previous
import functools

import jax
import jax.numpy as jnp
from jax import lax
from jax.experimental import pallas as pl
from jax.experimental.pallas import tpu as pltpu
from jax.experimental.pallas import tpu_sc as plsc

B = 8
N = 2048
L = 16
NC = 2
NS = 16
NW = NC * NS
BPC = B // NC
WPB = NS // BPC
SLICE = N // WPB
CH = 4
GRP = CH * L
BIG = 3.0e38
EPS = 1e-12


def _sqrt16(x):
    i = plsc.bitcast(x, jnp.int32)
    i = jnp.int32(0x5F3759DF) - (i >> 1)
    y = plsc.bitcast(i, jnp.float32)
    xh = x * jnp.float32(0.5)
    for _ in range(3):
        y = y * (jnp.float32(1.5) - xh * y * y)
    return x * y


def _cd_body(px_h, py_h, pz_h, pxb_h, pyb_h, pzb_h,
             qx_h, qy_h, qz_h, qxb_h, qyb_h, qzb_h,
             out1_h, out2_h,
             pxb_v, pyb_v, pzb_v, p2_v,
             qxb_v, qyb_v, qzb_v, q2_v,
             tx_v, ty_v, tz_v, comb_v, s_v, sp_d2):
    c = lax.axis_index("c")
    s = lax.axis_index("s")
    wid = c * NS + s
    b = c * BPC + s // WPB
    qtr = s % WPB
    pbase = b * N + qtr * SLICE
    qbase = b * N

    pltpu.sync_copy(pxb_h.at[pl.ds(pbase, SLICE)], pxb_v)
    pltpu.sync_copy(pyb_h.at[pl.ds(pbase, SLICE)], pyb_v)
    pltpu.sync_copy(pzb_h.at[pl.ds(pbase, SLICE)], pzb_v)
    pltpu.sync_copy(qxb_h.at[pl.ds(qbase, N)], qxb_v)
    pltpu.sync_copy(qyb_h.at[pl.ds(qbase, N)], qyb_v)
    pltpu.sync_copy(qzb_h.at[pl.ds(qbase, N)], qzb_v)

    pltpu.sync_copy(qx_h.at[pl.ds(qbase, N)], tx_v)
    pltpu.sync_copy(qy_h.at[pl.ds(qbase, N)], ty_v)
    pltpu.sync_copy(qz_h.at[pl.ds(qbase, N)], tz_v)
    for ch in range(N // L):
        x = tx_v[pl.ds(ch * L, L)]
        y = ty_v[pl.ds(ch * L, L)]
        z = tz_v[pl.ds(ch * L, L)]
        q2_v[pl.ds(ch * L, L)] = x * x + y * y + z * z
    pltpu.sync_copy(px_h.at[pl.ds(pbase, SLICE)], tx_v.at[pl.ds(0, SLICE)])
    pltpu.sync_copy(py_h.at[pl.ds(pbase, SLICE)], ty_v.at[pl.ds(0, SLICE)])
    pltpu.sync_copy(pz_h.at[pl.ds(pbase, SLICE)], tz_v.at[pl.ds(0, SLICE)])
    for ch in range(SLICE // L):
        x = tx_v[pl.ds(ch * L, L)]
        y = ty_v[pl.ds(ch * L, L)]
        z = tz_v[pl.ds(ch * L, L)]
        p2_v[pl.ds(ch * L, L)] = x * x + y * y + z * z

    s1 = jnp.zeros((L,), jnp.float32)
    for g in range(SLICE // GRP):
        base = g * GRP
        m2 = jnp.float32(-2.0)
        pxs = [pxb_v[pl.ds(base + k * L, L)] * m2 for k in range(CH)]
        pys = [pyb_v[pl.ds(base + k * L, L)] * m2 for k in range(CH)]
        pzs = [pzb_v[pl.ds(base + k * L, L)] * m2 for k in range(CH)]
        p2s = [p2_v[pl.ds(base + k * L, L)] for k in range(CH)]
        init = tuple(jnp.full((L,), BIG, jnp.float32) for _ in range(CH))

        @plsc.parallel_loop(0, N, unroll=4, carry=init)
        def acc1(j, acc):
            idx = jnp.full((L,), j, jnp.int32)
            qx = plsc.load_gather(qxb_v, [idx])
            qy = plsc.load_gather(qyb_v, [idx])
            qz = plsc.load_gather(qzb_v, [idx])
            q2 = plsc.load_gather(q2_v, [idx])
            out = []
            for k in range(CH):
                d = p2s[k] + (q2 + pxs[k] * qx + pys[k] * qy + pzs[k] * qz)
                out.append(jnp.minimum(acc[k], d))
            return tuple(out)

        for k in range(CH):
            s1 = s1 + _sqrt16(jnp.maximum(acc1[k], EPS))
    s_v[...] = s1
    pltpu.sync_copy(s_v, out1_h.at[wid])

    for g in range(N // GRP):
        base = g * GRP
        m2 = jnp.float32(-2.0)
        qxs = [qxb_v[pl.ds(base + k * L, L)] * m2 for k in range(CH)]
        qys = [qyb_v[pl.ds(base + k * L, L)] * m2 for k in range(CH)]
        qzs = [qzb_v[pl.ds(base + k * L, L)] * m2 for k in range(CH)]
        q2s = [q2_v[pl.ds(base + k * L, L)] for k in range(CH)]
        init = tuple(jnp.full((L,), BIG, jnp.float32) for _ in range(CH))

        @plsc.parallel_loop(0, SLICE, unroll=4, carry=init)
        def acc2(i, acc):
            idx = jnp.full((L,), i, jnp.int32)
            px = plsc.load_gather(pxb_v, [idx])
            py = plsc.load_gather(pyb_v, [idx])
            pz = plsc.load_gather(pzb_v, [idx])
            p2 = plsc.load_gather(p2_v, [idx])
            out = []
            for k in range(CH):
                d = q2s[k] + (p2 + qxs[k] * px + qys[k] * py + qzs[k] * pz)
                out.append(jnp.minimum(acc[k], d))
            return tuple(out)

        for k in range(CH):
            qxb_v[pl.ds(base + k * L, L)] = acc2[k]
    pltpu.sync_copy(qxb_v, sp_d2.at[s])
    plsc.subcore_barrier()

    s0 = (s // WPB) * WPB
    off = qtr * SLICE
    for t in range(WPB):
        pltpu.sync_copy(sp_d2.at[s0 + t, pl.ds(off, SLICE)], comb_v.at[t])
    s2 = jnp.zeros((L,), jnp.float32)
    for ch in range(SLICE // L):
        d = comb_v[0, pl.ds(ch * L, L)]
        for t in range(1, WPB):
            d = jnp.minimum(d, comb_v[t, pl.ds(ch * L, L)])
        s2 = s2 + _sqrt16(jnp.maximum(d, EPS))
    s_v[...] = s2
    pltpu.sync_copy(s_v, out2_h.at[wid])


_cd_call = pl.kernel(
    _cd_body,
    out_type=[
        jax.ShapeDtypeStruct((NW, L), jnp.float32),
        jax.ShapeDtypeStruct((NW, L), jnp.float32),
    ],
    mesh=plsc.VectorSubcoreMesh(
        core_axis_name="c", subcore_axis_name="s",
        num_cores=NC, num_subcores=NS),
    scratch_types=[
        pltpu.VMEM((SLICE,), jnp.float32),
        pltpu.VMEM((SLICE,), jnp.float32),
        pltpu.VMEM((SLICE,), jnp.float32),
        pltpu.VMEM((SLICE,), jnp.float32),
        pltpu.VMEM((N,), jnp.float32),
        pltpu.VMEM((N,), jnp.float32),
        pltpu.VMEM((N,), jnp.float32),
        pltpu.VMEM((N,), jnp.float32),
        pltpu.VMEM((N,), jnp.float32),
        pltpu.VMEM((N,), jnp.float32),
        pltpu.VMEM((N,), jnp.float32),
        pltpu.VMEM((WPB, SLICE), jnp.float32),
        pltpu.VMEM((L,), jnp.float32),
        pltpu.VMEM_SHARED((NS, N), jnp.float32),
    ],
    compiler_params=pltpu.CompilerParams(needs_layout_passes=False),
)


def _tc_body(pnc_ref, qt_ref, pbt_ref, qbt_ref, out_ref):
    p3c = pnc_ref[0]
    q3 = qt_ref[0]
    pb3 = pbt_ref[0].astype(jnp.bfloat16)
    qb3 = qbt_ref[0].astype(jnp.bfloat16)
    p2 = jnp.sum(p3c * p3c, axis=1, keepdims=True)
    q2 = jnp.sum(q3 * q3, axis=0, keepdims=True)
    cross = lax.dot_general(pb3, qb3, (((0,), (0,)), ((), ())),
                            preferred_element_type=jnp.float32)
    d = (p2 + q2) - (cross + cross)
    d = jnp.maximum(d, 1e-12)
    out_ref[0, 0, :] = jnp.sqrt(jnp.min(d, axis=1))
    out_ref[0, 1, :] = jnp.sqrt(jnp.min(d, axis=0))


def _make_tc_call(nb):
    return pl.pallas_call(
        _tc_body,
        grid=(nb,),
        in_specs=[
            pl.BlockSpec((1, N, 3), lambda b: (b, 0, 0)),
            pl.BlockSpec((1, 3, N), lambda b: (b, 0, 0)),
            pl.BlockSpec((1, 3, N), lambda b: (b, 0, 0)),
            pl.BlockSpec((1, 3, N), lambda b: (b, 0, 0)),
        ],
        out_specs=pl.BlockSpec((1, 2, N), lambda b: (b, 0, 0)),
        out_shape=jax.ShapeDtypeStruct((nb, 2, N), jnp.float32),
    )


_tc_call = _make_tc_call(B)


def _round_bf16_bits(x):
    i = lax.bitcast_convert_type(x, jnp.int32)
    r = (i + jnp.int32(0x7FFF) + ((i >> 16) & jnp.int32(1))) & jnp.int32(-65536)
    return lax.bitcast_convert_type(r, jnp.float32)


def kernel(y_pred, y_true):
    pt = jnp.transpose(y_pred, (0, 2, 1))
    qt = jnp.transpose(y_true, (0, 2, 1))
    pbt = _round_bf16_bits(pt)
    qbt = _round_bf16_bits(qt)
    sq = _tc_call(y_pred, qt, pbt, qbt)
    return jnp.mean(sq, axis=(0, 2)).sum() * 0.5


def _sc_kernel(y_pred, y_true):
    p = jnp.transpose(y_pred, (2, 0, 1)).reshape(3, B * N)
    q = jnp.transpose(y_true, (2, 0, 1)).reshape(3, B * N)
    def _rb(x):
        i = lax.bitcast_convert_type(x, jnp.int32)
        r = (i + jnp.int32(0x7FFF) + ((i >> 16) & jnp.int32(1))) & jnp.int32(-65536)
        return lax.bitcast_convert_type(r, jnp.float32)

    pb = _rb(p)
    qb = _rb(q)
    out1, out2 = _cd_call(p[0], p[1], p[2], pb[0], pb[1], pb[2],
                          q[0], q[1], q[2], qb[0], qb[1], qb[2])
    inv = 1.0 / (B * N)
    return (jnp.sum(out1) * inv + jnp.sum(out2) * inv) * 0.5


# speedup vs baseline: 4.8815x; 1.1200x over previous
import functools

import jax
import jax.numpy as jnp
from jax import lax
from jax.experimental import pallas as pl
from jax.experimental.pallas import tpu as pltpu
from jax.experimental.pallas import tpu_sc as plsc

B = 8
N = 2048
L = 16
NC = 2
NS = 16
NW = NC * NS
BPC = B // NC
WPB = NS // BPC
SLICE = N // WPB
CH = 4
GRP = CH * L
BIG = 3.0e38
EPS = 1e-12


def _sqrt16(x):
    i = plsc.bitcast(x, jnp.int32)
    i = jnp.int32(0x5F3759DF) - (i >> 1)
    y = plsc.bitcast(i, jnp.float32)
    xh = x * jnp.float32(0.5)
    for _ in range(3):
        y = y * (jnp.float32(1.5) - xh * y * y)
    return x * y


def _cd_body(px_h, py_h, pz_h, pxb_h, pyb_h, pzb_h,
             qx_h, qy_h, qz_h, qxb_h, qyb_h, qzb_h,
             out1_h, out2_h,
             pxb_v, pyb_v, pzb_v, p2_v,
             qxb_v, qyb_v, qzb_v, q2_v,
             tx_v, ty_v, tz_v, comb_v, s_v, sp_d2):
    c = lax.axis_index("c")
    s = lax.axis_index("s")
    wid = c * NS + s
    b = c * BPC + s // WPB
    qtr = s % WPB
    pbase = b * N + qtr * SLICE
    qbase = b * N

    pltpu.sync_copy(pxb_h.at[pl.ds(pbase, SLICE)], pxb_v)
    pltpu.sync_copy(pyb_h.at[pl.ds(pbase, SLICE)], pyb_v)
    pltpu.sync_copy(pzb_h.at[pl.ds(pbase, SLICE)], pzb_v)
    pltpu.sync_copy(qxb_h.at[pl.ds(qbase, N)], qxb_v)
    pltpu.sync_copy(qyb_h.at[pl.ds(qbase, N)], qyb_v)
    pltpu.sync_copy(qzb_h.at[pl.ds(qbase, N)], qzb_v)

    pltpu.sync_copy(qx_h.at[pl.ds(qbase, N)], tx_v)
    pltpu.sync_copy(qy_h.at[pl.ds(qbase, N)], ty_v)
    pltpu.sync_copy(qz_h.at[pl.ds(qbase, N)], tz_v)
    for ch in range(N // L):
        x = tx_v[pl.ds(ch * L, L)]
        y = ty_v[pl.ds(ch * L, L)]
        z = tz_v[pl.ds(ch * L, L)]
        q2_v[pl.ds(ch * L, L)] = x * x + y * y + z * z
    pltpu.sync_copy(px_h.at[pl.ds(pbase, SLICE)], tx_v.at[pl.ds(0, SLICE)])
    pltpu.sync_copy(py_h.at[pl.ds(pbase, SLICE)], ty_v.at[pl.ds(0, SLICE)])
    pltpu.sync_copy(pz_h.at[pl.ds(pbase, SLICE)], tz_v.at[pl.ds(0, SLICE)])
    for ch in range(SLICE // L):
        x = tx_v[pl.ds(ch * L, L)]
        y = ty_v[pl.ds(ch * L, L)]
        z = tz_v[pl.ds(ch * L, L)]
        p2_v[pl.ds(ch * L, L)] = x * x + y * y + z * z

    s1 = jnp.zeros((L,), jnp.float32)
    for g in range(SLICE // GRP):
        base = g * GRP
        m2 = jnp.float32(-2.0)
        pxs = [pxb_v[pl.ds(base + k * L, L)] * m2 for k in range(CH)]
        pys = [pyb_v[pl.ds(base + k * L, L)] * m2 for k in range(CH)]
        pzs = [pzb_v[pl.ds(base + k * L, L)] * m2 for k in range(CH)]
        p2s = [p2_v[pl.ds(base + k * L, L)] for k in range(CH)]
        init = tuple(jnp.full((L,), BIG, jnp.float32) for _ in range(CH))

        @plsc.parallel_loop(0, N, unroll=4, carry=init)
        def acc1(j, acc):
            idx = jnp.full((L,), j, jnp.int32)
            qx = plsc.load_gather(qxb_v, [idx])
            qy = plsc.load_gather(qyb_v, [idx])
            qz = plsc.load_gather(qzb_v, [idx])
            q2 = plsc.load_gather(q2_v, [idx])
            out = []
            for k in range(CH):
                d = p2s[k] + (q2 + pxs[k] * qx + pys[k] * qy + pzs[k] * qz)
                out.append(jnp.minimum(acc[k], d))
            return tuple(out)

        for k in range(CH):
            s1 = s1 + _sqrt16(jnp.maximum(acc1[k], EPS))
    s_v[...] = s1
    pltpu.sync_copy(s_v, out1_h.at[wid])

    for g in range(N // GRP):
        base = g * GRP
        m2 = jnp.float32(-2.0)
        qxs = [qxb_v[pl.ds(base + k * L, L)] * m2 for k in range(CH)]
        qys = [qyb_v[pl.ds(base + k * L, L)] * m2 for k in range(CH)]
        qzs = [qzb_v[pl.ds(base + k * L, L)] * m2 for k in range(CH)]
        q2s = [q2_v[pl.ds(base + k * L, L)] for k in range(CH)]
        init = tuple(jnp.full((L,), BIG, jnp.float32) for _ in range(CH))

        @plsc.parallel_loop(0, SLICE, unroll=4, carry=init)
        def acc2(i, acc):
            idx = jnp.full((L,), i, jnp.int32)
            px = plsc.load_gather(pxb_v, [idx])
            py = plsc.load_gather(pyb_v, [idx])
            pz = plsc.load_gather(pzb_v, [idx])
            p2 = plsc.load_gather(p2_v, [idx])
            out = []
            for k in range(CH):
                d = q2s[k] + (p2 + qxs[k] * px + qys[k] * py + qzs[k] * pz)
                out.append(jnp.minimum(acc[k], d))
            return tuple(out)

        for k in range(CH):
            qxb_v[pl.ds(base + k * L, L)] = acc2[k]
    pltpu.sync_copy(qxb_v, sp_d2.at[s])
    plsc.subcore_barrier()

    s0 = (s // WPB) * WPB
    off = qtr * SLICE
    for t in range(WPB):
        pltpu.sync_copy(sp_d2.at[s0 + t, pl.ds(off, SLICE)], comb_v.at[t])
    s2 = jnp.zeros((L,), jnp.float32)
    for ch in range(SLICE // L):
        d = comb_v[0, pl.ds(ch * L, L)]
        for t in range(1, WPB):
            d = jnp.minimum(d, comb_v[t, pl.ds(ch * L, L)])
        s2 = s2 + _sqrt16(jnp.maximum(d, EPS))
    s_v[...] = s2
    pltpu.sync_copy(s_v, out2_h.at[wid])


_cd_call = pl.kernel(
    _cd_body,
    out_type=[
        jax.ShapeDtypeStruct((NW, L), jnp.float32),
        jax.ShapeDtypeStruct((NW, L), jnp.float32),
    ],
    mesh=plsc.VectorSubcoreMesh(
        core_axis_name="c", subcore_axis_name="s",
        num_cores=NC, num_subcores=NS),
    scratch_types=[
        pltpu.VMEM((SLICE,), jnp.float32),
        pltpu.VMEM((SLICE,), jnp.float32),
        pltpu.VMEM((SLICE,), jnp.float32),
        pltpu.VMEM((SLICE,), jnp.float32),
        pltpu.VMEM((N,), jnp.float32),
        pltpu.VMEM((N,), jnp.float32),
        pltpu.VMEM((N,), jnp.float32),
        pltpu.VMEM((N,), jnp.float32),
        pltpu.VMEM((N,), jnp.float32),
        pltpu.VMEM((N,), jnp.float32),
        pltpu.VMEM((N,), jnp.float32),
        pltpu.VMEM((WPB, SLICE), jnp.float32),
        pltpu.VMEM((L,), jnp.float32),
        pltpu.VMEM_SHARED((NS, N), jnp.float32),
    ],
    compiler_params=pltpu.CompilerParams(needs_layout_passes=False),
)


def _tc_body(pnc_ref, qt_ref, pbt_ref, qbt_ref, out_ref):
    p3c = pnc_ref[0]
    q3 = qt_ref[0]
    pb3 = pbt_ref[0].astype(jnp.bfloat16)
    qb3 = qbt_ref[0].astype(jnp.bfloat16)
    p2 = jnp.sum(p3c * p3c, axis=1, keepdims=True)
    q2 = jnp.sum(q3 * q3, axis=0, keepdims=True)
    cross2 = lax.dot_general(pb3, qb3, (((0,), (0,)), ((), ())),
                             preferred_element_type=jnp.float32)
    d = p2 + (q2 + cross2)
    out_ref[0, 0, :] = jnp.sqrt(jnp.maximum(jnp.min(d, axis=1), 1e-12))
    out_ref[0, 1, :] = jnp.sqrt(jnp.maximum(jnp.min(d, axis=0), 1e-12))


def _make_tc_call(nb):
    return pl.pallas_call(
        _tc_body,
        grid=(nb,),
        in_specs=[
            pl.BlockSpec((1, N, 3), lambda b: (b, 0, 0)),
            pl.BlockSpec((1, 3, N), lambda b: (b, 0, 0)),
            pl.BlockSpec((1, 3, N), lambda b: (b, 0, 0)),
            pl.BlockSpec((1, 3, N), lambda b: (b, 0, 0)),
        ],
        out_specs=pl.BlockSpec((1, 2, N), lambda b: (b, 0, 0)),
        out_shape=jax.ShapeDtypeStruct((nb, 2, N), jnp.float32),
    )


_tc_call = _make_tc_call(B)


def _round_bf16_bits(x):
    i = lax.bitcast_convert_type(x, jnp.int32)
    r = (i + jnp.int32(0x7FFF) + ((i >> 16) & jnp.int32(1))) & jnp.int32(-65536)
    return lax.bitcast_convert_type(r, jnp.float32)


def kernel(y_pred, y_true):
    pt = jnp.transpose(y_pred, (0, 2, 1))
    qt = jnp.transpose(y_true, (0, 2, 1))
    pbt = _round_bf16_bits(pt) * jnp.float32(-2.0)
    qbt = _round_bf16_bits(qt)
    sq = _tc_call(y_pred, qt, pbt, qbt)
    return jnp.mean(sq, axis=(0, 2)).sum() * 0.5


def _sc_kernel(y_pred, y_true):
    p = jnp.transpose(y_pred, (2, 0, 1)).reshape(3, B * N)
    q = jnp.transpose(y_true, (2, 0, 1)).reshape(3, B * N)
    def _rb(x):
        i = lax.bitcast_convert_type(x, jnp.int32)
        r = (i + jnp.int32(0x7FFF) + ((i >> 16) & jnp.int32(1))) & jnp.int32(-65536)
        return lax.bitcast_convert_type(r, jnp.float32)

    pb = _rb(p)
    qb = _rb(q)
    out1, out2 = _cd_call(p[0], p[1], p[2], pb[0], pb[1], pb[2],
                          q[0], q[1], q[2], qb[0], qb[1], qb[2])
    inv = 1.0 / (B * N)
    return (jnp.sum(out1) * inv + jnp.sum(out2) * inv) * 0.5


# speedup vs baseline: 4.9585x; 1.0158x over previous
import functools

import jax
import jax.numpy as jnp
from jax import lax
from jax.experimental import pallas as pl
from jax.experimental.pallas import tpu as pltpu
from jax.experimental.pallas import tpu_sc as plsc

B = 8
N = 2048
L = 16
NC = 2
NS = 16
NW = NC * NS
BPC = B // NC
WPB = NS // BPC
SLICE = N // WPB
CH = 4
GRP = CH * L
BIG = 3.0e38
EPS = 1e-12


def _sqrt16(x):
    i = plsc.bitcast(x, jnp.int32)
    i = jnp.int32(0x5F3759DF) - (i >> 1)
    y = plsc.bitcast(i, jnp.float32)
    xh = x * jnp.float32(0.5)
    for _ in range(3):
        y = y * (jnp.float32(1.5) - xh * y * y)
    return x * y


def _cd_body(px_h, py_h, pz_h, pxb_h, pyb_h, pzb_h,
             qx_h, qy_h, qz_h, qxb_h, qyb_h, qzb_h,
             out1_h, out2_h,
             pxb_v, pyb_v, pzb_v, p2_v,
             qxb_v, qyb_v, qzb_v, q2_v,
             tx_v, ty_v, tz_v, comb_v, s_v, sp_d2):
    c = lax.axis_index("c")
    s = lax.axis_index("s")
    wid = c * NS + s
    b = c * BPC + s // WPB
    qtr = s % WPB
    pbase = b * N + qtr * SLICE
    qbase = b * N

    pltpu.sync_copy(pxb_h.at[pl.ds(pbase, SLICE)], pxb_v)
    pltpu.sync_copy(pyb_h.at[pl.ds(pbase, SLICE)], pyb_v)
    pltpu.sync_copy(pzb_h.at[pl.ds(pbase, SLICE)], pzb_v)
    pltpu.sync_copy(qxb_h.at[pl.ds(qbase, N)], qxb_v)
    pltpu.sync_copy(qyb_h.at[pl.ds(qbase, N)], qyb_v)
    pltpu.sync_copy(qzb_h.at[pl.ds(qbase, N)], qzb_v)

    pltpu.sync_copy(qx_h.at[pl.ds(qbase, N)], tx_v)
    pltpu.sync_copy(qy_h.at[pl.ds(qbase, N)], ty_v)
    pltpu.sync_copy(qz_h.at[pl.ds(qbase, N)], tz_v)
    for ch in range(N // L):
        x = tx_v[pl.ds(ch * L, L)]
        y = ty_v[pl.ds(ch * L, L)]
        z = tz_v[pl.ds(ch * L, L)]
        q2_v[pl.ds(ch * L, L)] = x * x + y * y + z * z
    pltpu.sync_copy(px_h.at[pl.ds(pbase, SLICE)], tx_v.at[pl.ds(0, SLICE)])
    pltpu.sync_copy(py_h.at[pl.ds(pbase, SLICE)], ty_v.at[pl.ds(0, SLICE)])
    pltpu.sync_copy(pz_h.at[pl.ds(pbase, SLICE)], tz_v.at[pl.ds(0, SLICE)])
    for ch in range(SLICE // L):
        x = tx_v[pl.ds(ch * L, L)]
        y = ty_v[pl.ds(ch * L, L)]
        z = tz_v[pl.ds(ch * L, L)]
        p2_v[pl.ds(ch * L, L)] = x * x + y * y + z * z

    s1 = jnp.zeros((L,), jnp.float32)
    for g in range(SLICE // GRP):
        base = g * GRP
        m2 = jnp.float32(-2.0)
        pxs = [pxb_v[pl.ds(base + k * L, L)] * m2 for k in range(CH)]
        pys = [pyb_v[pl.ds(base + k * L, L)] * m2 for k in range(CH)]
        pzs = [pzb_v[pl.ds(base + k * L, L)] * m2 for k in range(CH)]
        p2s = [p2_v[pl.ds(base + k * L, L)] for k in range(CH)]
        init = tuple(jnp.full((L,), BIG, jnp.float32) for _ in range(CH))

        @plsc.parallel_loop(0, N, unroll=4, carry=init)
        def acc1(j, acc):
            idx = jnp.full((L,), j, jnp.int32)
            qx = plsc.load_gather(qxb_v, [idx])
            qy = plsc.load_gather(qyb_v, [idx])
            qz = plsc.load_gather(qzb_v, [idx])
            q2 = plsc.load_gather(q2_v, [idx])
            out = []
            for k in range(CH):
                d = p2s[k] + (q2 + pxs[k] * qx + pys[k] * qy + pzs[k] * qz)
                out.append(jnp.minimum(acc[k], d))
            return tuple(out)

        for k in range(CH):
            s1 = s1 + _sqrt16(jnp.maximum(acc1[k], EPS))
    s_v[...] = s1
    pltpu.sync_copy(s_v, out1_h.at[wid])

    for g in range(N // GRP):
        base = g * GRP
        m2 = jnp.float32(-2.0)
        qxs = [qxb_v[pl.ds(base + k * L, L)] * m2 for k in range(CH)]
        qys = [qyb_v[pl.ds(base + k * L, L)] * m2 for k in range(CH)]
        qzs = [qzb_v[pl.ds(base + k * L, L)] * m2 for k in range(CH)]
        q2s = [q2_v[pl.ds(base + k * L, L)] for k in range(CH)]
        init = tuple(jnp.full((L,), BIG, jnp.float32) for _ in range(CH))

        @plsc.parallel_loop(0, SLICE, unroll=4, carry=init)
        def acc2(i, acc):
            idx = jnp.full((L,), i, jnp.int32)
            px = plsc.load_gather(pxb_v, [idx])
            py = plsc.load_gather(pyb_v, [idx])
            pz = plsc.load_gather(pzb_v, [idx])
            p2 = plsc.load_gather(p2_v, [idx])
            out = []
            for k in range(CH):
                d = q2s[k] + (p2 + qxs[k] * px + qys[k] * py + qzs[k] * pz)
                out.append(jnp.minimum(acc[k], d))
            return tuple(out)

        for k in range(CH):
            qxb_v[pl.ds(base + k * L, L)] = acc2[k]
    pltpu.sync_copy(qxb_v, sp_d2.at[s])
    plsc.subcore_barrier()

    s0 = (s // WPB) * WPB
    off = qtr * SLICE
    for t in range(WPB):
        pltpu.sync_copy(sp_d2.at[s0 + t, pl.ds(off, SLICE)], comb_v.at[t])
    s2 = jnp.zeros((L,), jnp.float32)
    for ch in range(SLICE // L):
        d = comb_v[0, pl.ds(ch * L, L)]
        for t in range(1, WPB):
            d = jnp.minimum(d, comb_v[t, pl.ds(ch * L, L)])
        s2 = s2 + _sqrt16(jnp.maximum(d, EPS))
    s_v[...] = s2
    pltpu.sync_copy(s_v, out2_h.at[wid])


_cd_call = pl.kernel(
    _cd_body,
    out_type=[
        jax.ShapeDtypeStruct((NW, L), jnp.float32),
        jax.ShapeDtypeStruct((NW, L), jnp.float32),
    ],
    mesh=plsc.VectorSubcoreMesh(
        core_axis_name="c", subcore_axis_name="s",
        num_cores=NC, num_subcores=NS),
    scratch_types=[
        pltpu.VMEM((SLICE,), jnp.float32),
        pltpu.VMEM((SLICE,), jnp.float32),
        pltpu.VMEM((SLICE,), jnp.float32),
        pltpu.VMEM((SLICE,), jnp.float32),
        pltpu.VMEM((N,), jnp.float32),
        pltpu.VMEM((N,), jnp.float32),
        pltpu.VMEM((N,), jnp.float32),
        pltpu.VMEM((N,), jnp.float32),
        pltpu.VMEM((N,), jnp.float32),
        pltpu.VMEM((N,), jnp.float32),
        pltpu.VMEM((N,), jnp.float32),
        pltpu.VMEM((WPB, SLICE), jnp.float32),
        pltpu.VMEM((L,), jnp.float32),
        pltpu.VMEM_SHARED((NS, N), jnp.float32),
    ],
    compiler_params=pltpu.CompilerParams(needs_layout_passes=False),
)


def _tc_body(pnc_ref, pt_ref, qnc_ref, qt_ref, pbt_ref, qbt_ref, out_ref):
    p3c = pnc_ref[0]
    p3r = pt_ref[0]
    q3c = qnc_ref[0]
    q3r = qt_ref[0]
    pb3 = pbt_ref[0].astype(jnp.bfloat16)
    qb3 = qbt_ref[0].astype(jnp.bfloat16)
    p2c = jnp.sum(p3c * p3c, axis=1, keepdims=True)
    q2c = jnp.sum(q3c * q3c, axis=1, keepdims=True)
    p2r = jnp.sum(p3r * p3r, axis=0, keepdims=True)
    q2r = jnp.sum(q3r * q3r, axis=0, keepdims=True)
    cr = lax.dot_general(pb3, qb3, (((0,), (0,)), ((), ())),
                         preferred_element_type=jnp.float32)
    crT = lax.dot_general(qb3, pb3, (((0,), (0,)), ((), ())),
                          preferred_element_type=jnp.float32)
    d = p2c + (q2r + cr)
    dT = q2c + (p2r + crT)
    out_ref[0, 0, :] = jnp.sqrt(jnp.maximum(jnp.min(dT, axis=0), 1e-12))
    out_ref[0, 1, :] = jnp.sqrt(jnp.maximum(jnp.min(d, axis=0), 1e-12))


def _make_tc_call(nb):
    return pl.pallas_call(
        _tc_body,
        grid=(nb,),
        in_specs=[
            pl.BlockSpec((1, N, 3), lambda b: (b, 0, 0)),
            pl.BlockSpec((1, 3, N), lambda b: (b, 0, 0)),
            pl.BlockSpec((1, N, 3), lambda b: (b, 0, 0)),
            pl.BlockSpec((1, 3, N), lambda b: (b, 0, 0)),
            pl.BlockSpec((1, 3, N), lambda b: (b, 0, 0)),
            pl.BlockSpec((1, 3, N), lambda b: (b, 0, 0)),
        ],
        out_specs=pl.BlockSpec((1, 2, N), lambda b: (b, 0, 0)),
        out_shape=jax.ShapeDtypeStruct((nb, 2, N), jnp.float32),
    )


_tc_call = _make_tc_call(B)


def _round_bf16_bits(x):
    i = lax.bitcast_convert_type(x, jnp.int32)
    r = (i + jnp.int32(0x7FFF) + ((i >> 16) & jnp.int32(1))) & jnp.int32(-65536)
    return lax.bitcast_convert_type(r, jnp.float32)


def kernel(y_pred, y_true):
    pt = jnp.transpose(y_pred, (0, 2, 1))
    qt = jnp.transpose(y_true, (0, 2, 1))
    pbt = _round_bf16_bits(pt) * jnp.float32(-2.0)
    qbt = _round_bf16_bits(qt)
    sq = _tc_call(y_pred, pt, y_true, qt, pbt, qbt)
    return jnp.mean(sq, axis=(0, 2)).sum() * 0.5


def _sc_kernel(y_pred, y_true):
    p = jnp.transpose(y_pred, (2, 0, 1)).reshape(3, B * N)
    q = jnp.transpose(y_true, (2, 0, 1)).reshape(3, B * N)
    def _rb(x):
        i = lax.bitcast_convert_type(x, jnp.int32)
        r = (i + jnp.int32(0x7FFF) + ((i >> 16) & jnp.int32(1))) & jnp.int32(-65536)
        return lax.bitcast_convert_type(r, jnp.float32)

    pb = _rb(p)
    qb = _rb(q)
    out1, out2 = _cd_call(p[0], p[1], p[2], pb[0], pb[1], pb[2],
                          q[0], q[1], q[2], qb[0], qb[1], qb[2])
    inv = 1.0 / (B * N)
    return (jnp.sum(out1) * inv + jnp.sum(out2) * inv) * 0.5


# speedup vs baseline: 5.0718x; 1.0228x over previous
import functools

import jax
import jax.numpy as jnp
from jax import lax
from jax.experimental import pallas as pl
from jax.experimental.pallas import tpu as pltpu
from jax.experimental.pallas import tpu_sc as plsc

B = 8
N = 2048
L = 16
NC = 2
NS = 16
NW = NC * NS
BPC = B // NC
WPB = NS // BPC
SLICE = N // WPB
CH = 4
GRP = CH * L
BIG = 3.0e38
EPS = 1e-12


def _sqrt16(x):
    i = plsc.bitcast(x, jnp.int32)
    i = jnp.int32(0x5F3759DF) - (i >> 1)
    y = plsc.bitcast(i, jnp.float32)
    xh = x * jnp.float32(0.5)
    for _ in range(3):
        y = y * (jnp.float32(1.5) - xh * y * y)
    return x * y


def _cd_body(px_h, py_h, pz_h, pxb_h, pyb_h, pzb_h,
             qx_h, qy_h, qz_h, qxb_h, qyb_h, qzb_h,
             out1_h, out2_h,
             pxb_v, pyb_v, pzb_v, p2_v,
             qxb_v, qyb_v, qzb_v, q2_v,
             tx_v, ty_v, tz_v, comb_v, s_v, sp_d2):
    c = lax.axis_index("c")
    s = lax.axis_index("s")
    wid = c * NS + s
    b = c * BPC + s // WPB
    qtr = s % WPB
    pbase = b * N + qtr * SLICE
    qbase = b * N

    pltpu.sync_copy(pxb_h.at[pl.ds(pbase, SLICE)], pxb_v)
    pltpu.sync_copy(pyb_h.at[pl.ds(pbase, SLICE)], pyb_v)
    pltpu.sync_copy(pzb_h.at[pl.ds(pbase, SLICE)], pzb_v)
    pltpu.sync_copy(qxb_h.at[pl.ds(qbase, N)], qxb_v)
    pltpu.sync_copy(qyb_h.at[pl.ds(qbase, N)], qyb_v)
    pltpu.sync_copy(qzb_h.at[pl.ds(qbase, N)], qzb_v)

    pltpu.sync_copy(qx_h.at[pl.ds(qbase, N)], tx_v)
    pltpu.sync_copy(qy_h.at[pl.ds(qbase, N)], ty_v)
    pltpu.sync_copy(qz_h.at[pl.ds(qbase, N)], tz_v)
    for ch in range(N // L):
        x = tx_v[pl.ds(ch * L, L)]
        y = ty_v[pl.ds(ch * L, L)]
        z = tz_v[pl.ds(ch * L, L)]
        q2_v[pl.ds(ch * L, L)] = x * x + y * y + z * z
    pltpu.sync_copy(px_h.at[pl.ds(pbase, SLICE)], tx_v.at[pl.ds(0, SLICE)])
    pltpu.sync_copy(py_h.at[pl.ds(pbase, SLICE)], ty_v.at[pl.ds(0, SLICE)])
    pltpu.sync_copy(pz_h.at[pl.ds(pbase, SLICE)], tz_v.at[pl.ds(0, SLICE)])
    for ch in range(SLICE // L):
        x = tx_v[pl.ds(ch * L, L)]
        y = ty_v[pl.ds(ch * L, L)]
        z = tz_v[pl.ds(ch * L, L)]
        p2_v[pl.ds(ch * L, L)] = x * x + y * y + z * z

    s1 = jnp.zeros((L,), jnp.float32)
    for g in range(SLICE // GRP):
        base = g * GRP
        m2 = jnp.float32(-2.0)
        pxs = [pxb_v[pl.ds(base + k * L, L)] * m2 for k in range(CH)]
        pys = [pyb_v[pl.ds(base + k * L, L)] * m2 for k in range(CH)]
        pzs = [pzb_v[pl.ds(base + k * L, L)] * m2 for k in range(CH)]
        p2s = [p2_v[pl.ds(base + k * L, L)] for k in range(CH)]
        init = tuple(jnp.full((L,), BIG, jnp.float32) for _ in range(CH))

        @plsc.parallel_loop(0, N, unroll=4, carry=init)
        def acc1(j, acc):
            idx = jnp.full((L,), j, jnp.int32)
            qx = plsc.load_gather(qxb_v, [idx])
            qy = plsc.load_gather(qyb_v, [idx])
            qz = plsc.load_gather(qzb_v, [idx])
            q2 = plsc.load_gather(q2_v, [idx])
            out = []
            for k in range(CH):
                d = p2s[k] + (q2 + pxs[k] * qx + pys[k] * qy + pzs[k] * qz)
                out.append(jnp.minimum(acc[k], d))
            return tuple(out)

        for k in range(CH):
            s1 = s1 + _sqrt16(jnp.maximum(acc1[k], EPS))
    s_v[...] = s1
    pltpu.sync_copy(s_v, out1_h.at[wid])

    for g in range(N // GRP):
        base = g * GRP
        m2 = jnp.float32(-2.0)
        qxs = [qxb_v[pl.ds(base + k * L, L)] * m2 for k in range(CH)]
        qys = [qyb_v[pl.ds(base + k * L, L)] * m2 for k in range(CH)]
        qzs = [qzb_v[pl.ds(base + k * L, L)] * m2 for k in range(CH)]
        q2s = [q2_v[pl.ds(base + k * L, L)] for k in range(CH)]
        init = tuple(jnp.full((L,), BIG, jnp.float32) for _ in range(CH))

        @plsc.parallel_loop(0, SLICE, unroll=4, carry=init)
        def acc2(i, acc):
            idx = jnp.full((L,), i, jnp.int32)
            px = plsc.load_gather(pxb_v, [idx])
            py = plsc.load_gather(pyb_v, [idx])
            pz = plsc.load_gather(pzb_v, [idx])
            p2 = plsc.load_gather(p2_v, [idx])
            out = []
            for k in range(CH):
                d = q2s[k] + (p2 + qxs[k] * px + qys[k] * py + qzs[k] * pz)
                out.append(jnp.minimum(acc[k], d))
            return tuple(out)

        for k in range(CH):
            qxb_v[pl.ds(base + k * L, L)] = acc2[k]
    pltpu.sync_copy(qxb_v, sp_d2.at[s])
    plsc.subcore_barrier()

    s0 = (s // WPB) * WPB
    off = qtr * SLICE
    for t in range(WPB):
        pltpu.sync_copy(sp_d2.at[s0 + t, pl.ds(off, SLICE)], comb_v.at[t])
    s2 = jnp.zeros((L,), jnp.float32)
    for ch in range(SLICE // L):
        d = comb_v[0, pl.ds(ch * L, L)]
        for t in range(1, WPB):
            d = jnp.minimum(d, comb_v[t, pl.ds(ch * L, L)])
        s2 = s2 + _sqrt16(jnp.maximum(d, EPS))
    s_v[...] = s2
    pltpu.sync_copy(s_v, out2_h.at[wid])


_cd_call = pl.kernel(
    _cd_body,
    out_type=[
        jax.ShapeDtypeStruct((NW, L), jnp.float32),
        jax.ShapeDtypeStruct((NW, L), jnp.float32),
    ],
    mesh=plsc.VectorSubcoreMesh(
        core_axis_name="c", subcore_axis_name="s",
        num_cores=NC, num_subcores=NS),
    scratch_types=[
        pltpu.VMEM((SLICE,), jnp.float32),
        pltpu.VMEM((SLICE,), jnp.float32),
        pltpu.VMEM((SLICE,), jnp.float32),
        pltpu.VMEM((SLICE,), jnp.float32),
        pltpu.VMEM((N,), jnp.float32),
        pltpu.VMEM((N,), jnp.float32),
        pltpu.VMEM((N,), jnp.float32),
        pltpu.VMEM((N,), jnp.float32),
        pltpu.VMEM((N,), jnp.float32),
        pltpu.VMEM((N,), jnp.float32),
        pltpu.VMEM((N,), jnp.float32),
        pltpu.VMEM((WPB, SLICE), jnp.float32),
        pltpu.VMEM((L,), jnp.float32),
        pltpu.VMEM_SHARED((NS, N), jnp.float32),
    ],
    compiler_params=pltpu.CompilerParams(needs_layout_passes=False),
)


def _tc_body(pnc_ref, pt_ref, qnc_ref, qt_ref, pbt_ref, qbt_ref, out_ref):
    p3c = pnc_ref[0]
    p3r = pt_ref[0]
    q3c = qnc_ref[0]
    q3r = qt_ref[0]
    pb3 = pbt_ref[0].astype(jnp.bfloat16)
    qb3 = qbt_ref[0].astype(jnp.bfloat16)
    p2c = jnp.sum(p3c * p3c, axis=1, keepdims=True)
    q2c = jnp.sum(q3c * q3c, axis=1, keepdims=True)
    p2r = jnp.sum(p3r * p3r, axis=0, keepdims=True)
    q2r = jnp.sum(q3r * q3r, axis=0, keepdims=True)
    cr = lax.dot_general(pb3, qb3, (((0,), (0,)), ((), ())),
                         preferred_element_type=jnp.float32)
    crT = lax.dot_general(qb3, pb3, (((0,), (0,)), ((), ())),
                          preferred_element_type=jnp.float32)
    e = p2c + cr
    eT = q2c + crT
    d1 = jnp.min(eT, axis=0) + p2r[0]
    d2 = jnp.min(e, axis=0) + q2r[0]
    out_ref[0, 0, :] = jnp.sqrt(jnp.maximum(d1, 1e-12))
    out_ref[0, 1, :] = jnp.sqrt(jnp.maximum(d2, 1e-12))


def _make_tc_call(nb):
    return pl.pallas_call(
        _tc_body,
        grid=(nb,),
        in_specs=[
            pl.BlockSpec((1, N, 3), lambda b: (b, 0, 0)),
            pl.BlockSpec((1, 3, N), lambda b: (b, 0, 0)),
            pl.BlockSpec((1, N, 3), lambda b: (b, 0, 0)),
            pl.BlockSpec((1, 3, N), lambda b: (b, 0, 0)),
            pl.BlockSpec((1, 3, N), lambda b: (b, 0, 0)),
            pl.BlockSpec((1, 3, N), lambda b: (b, 0, 0)),
        ],
        out_specs=pl.BlockSpec((1, 2, N), lambda b: (b, 0, 0)),
        out_shape=jax.ShapeDtypeStruct((nb, 2, N), jnp.float32),
    )


_tc_call = _make_tc_call(B)


def _round_bf16_bits(x):
    i = lax.bitcast_convert_type(x, jnp.int32)
    r = (i + jnp.int32(0x7FFF) + ((i >> 16) & jnp.int32(1))) & jnp.int32(-65536)
    return lax.bitcast_convert_type(r, jnp.float32)


def kernel(y_pred, y_true):
    pt = jnp.transpose(y_pred, (0, 2, 1))
    qt = jnp.transpose(y_true, (0, 2, 1))
    pbt = _round_bf16_bits(pt) * jnp.float32(-2.0)
    qbt = _round_bf16_bits(qt)
    sq = _tc_call(y_pred, pt, y_true, qt, pbt, qbt)
    return jnp.mean(sq, axis=(0, 2)).sum() * 0.5


def _sc_kernel(y_pred, y_true):
    p = jnp.transpose(y_pred, (2, 0, 1)).reshape(3, B * N)
    q = jnp.transpose(y_true, (2, 0, 1)).reshape(3, B * N)
    def _rb(x):
        i = lax.bitcast_convert_type(x, jnp.int32)
        r = (i + jnp.int32(0x7FFF) + ((i >> 16) & jnp.int32(1))) & jnp.int32(-65536)
        return lax.bitcast_convert_type(r, jnp.float32)

    pb = _rb(p)
    qb = _rb(q)
    out1, out2 = _cd_call(p[0], p[1], p[2], pb[0], pb[1], pb[2],
                          q[0], q[1], q[2], qb[0], qb[1], qb[2])
    inv = 1.0 / (B * N)
    return (jnp.sum(out1) * inv + jnp.sum(out2) * inv) * 0.5
